# Initial kernel scaffold; baseline (speedup 1.0000x reference)
import functools

import jax
import jax.numpy as jnp
from jax import lax
from jax.experimental import pallas as pl
from jax.experimental.pallas import tpu as pltpu
from jax.experimental.pallas import tpu_sc as plsc

L = 16384
D = 32
PAD = 64
PL = L + 192
NLANE = 16


def _nms_body(x_hbm, out_hbm, xb, vb, kb, ab, bb):
    nc = 2
    wid = lax.axis_index("s") * nc + lax.axis_index("c")

    @pl.when(wid < 8)
    def _():
        b = wid
        pltpu.sync_copy(x_hbm.at[b, 0], xb.at[pl.ds(PAD, L)])

        neg1 = jnp.full((NLANE,), -1.0, dtype=jnp.float32)
        zero = jnp.zeros((NLANE,), dtype=jnp.float32)

        def init_ab(t, _):
            ab[pl.ds(t * NLANE, NLANE)] = neg1
            bb[pl.ds(t * NLANE, NLANE)] = neg1
            kb[pl.ds(t * NLANE, NLANE)] = zero
            return 0

        lax.fori_loop(0, PL // NLANE, init_ab, 0)

        def init_vpad_lo(t, _):
            vb[pl.ds(t * NLANE, NLANE)] = neg1
            return 0

        lax.fori_loop(0, PAD // NLANE, init_vpad_lo, 0)

        def init_vpad_hi(t, _):
            vb[pl.ds(PAD + L + t * NLANE, NLANE)] = neg1
            return 0

        lax.fori_loop(0, (PL - PAD - L) // NLANE, init_vpad_hi, 0)

        lane = lax.iota(jnp.int32, (NLANE,))

        def detect(t, _):
            base = PAD + t * NLANE
            ig = lane + t * NLANE
            xm = xb[pl.ds(base - 1, NLANE)]
            xc = xb[pl.ds(base, NLANE)]
            xp = xb[pl.ds(base + 1, NLANE)]
            dxr = ((xp - xc) > 0) & (ig < (L - 1))
            dxl = ((xc - xm) <= 0) | (ig == 0)
            valley = dxr & dxl & (xc <= 0)
            peak = (~dxr) & (~dxl) & (xc > 0)
            ex = valley | peak
            vb[pl.ds(base, NLANE)] = jnp.where(ex, jnp.abs(xc), -1.0)
            return 0

        lax.fori_loop(0, L // NLANE, detect, 0)

        niter = (L + 2 * D) // NLANE + 2

        def run64(src):
            bufs = [ab, bb]
            cur = src
            for p, s in enumerate((1, 2, 4, 8, 16, 32)):
                dst = bufs[p % 2]
                src_ref = cur

                def pass_body(t, _, src_ref=src_ref, dst=dst, s=s):
                    j = 2 * D + t * NLANE
                    a = src_ref[pl.ds(j, NLANE)]
                    c = src_ref[pl.ds(j + s, NLANE)]
                    dst[pl.ds(j, NLANE)] = jnp.maximum(a, c)
                    return 0

                lax.fori_loop(0, niter, pass_body, 0)
                cur = dst
            return cur

        def round_body(carry):
            _ = carry
            r64 = run64(vb)

            def keep(t, _):
                i0 = PAD + t * NLANE
                w = jnp.maximum(r64[pl.ds(i0 - D, NLANE)], vb[pl.ds(i0 + D, NLANE)])
                v = vb[pl.ds(i0, NLANE)]
                kn = (v >= 0) & (v >= w)
                kb[pl.ds(i0, NLANE)] = jnp.where(kn, 1.0, kb[pl.ds(i0, NLANE)])
                return 0

            lax.fori_loop(0, L // NLANE, keep, 0)

            r64k = run64(kb)

            def suppress(t, acc):
                i0 = PAD + t * NLANE
                s = jnp.maximum(r64k[pl.ds(i0 - D, NLANE)], kb[pl.ds(i0 + D, NLANE)])
                v = vb[pl.ds(i0, NLANE)]
                vn = jnp.where(s > 0, -1.0, v)
                vb[pl.ds(i0, NLANE)] = vn
                return jnp.maximum(acc, vn)

            acc = lax.fori_loop(0, L // NLANE, suppress, neg1)
            alive = lax.reduce_max(acc, axes=(0,)) >= 0
            return alive.astype(jnp.int32)

        lax.while_loop(lambda c: c > 0, round_body, jnp.int32(1))

        def emit(t, _):
            i0 = PAD + t * NLANE
            o = jnp.where(kb[pl.ds(i0, NLANE)] > 0, xb[pl.ds(i0, NLANE)], 0.0)
            ab[pl.ds(i0, NLANE)] = o
            return 0

        lax.fori_loop(0, L // NLANE, emit, 0)
        pltpu.sync_copy(ab.at[pl.ds(PAD, L)], out_hbm.at[b, 0])


@jax.jit
def _nms(input_):
    mesh = plsc.VectorSubcoreMesh(core_axis_name="c", subcore_axis_name="s")
    f32 = jnp.float32
    return pl.kernel(
        _nms_body,
        mesh=mesh,
        out_type=jax.ShapeDtypeStruct((8, 1, L), f32),
        scratch_types=[
            pltpu.VMEM((PL,), f32),
            pltpu.VMEM((PL,), f32),
            pltpu.VMEM((PL,), f32),
            pltpu.VMEM((PL,), f32),
            pltpu.VMEM((PL,), f32),
        ],
    )(input_)


def kernel(input_):
    return _nms(input_)


# speedup vs baseline: 434.1415x; 434.1415x over previous
import functools

import jax
import jax.numpy as jnp
from jax import lax
from jax.experimental import pallas as pl
from jax.experimental.pallas import tpu as pltpu
from jax.experimental.pallas import tpu_sc as plsc

L = 16384
D = 32
PAD = 64
PL = L + 192
NLANE = 16


def _nms_body(x_hbm, out_hbm, xb, vb, kb, ab, bb, flag):
    nc = 2
    wid = lax.axis_index("s") * nc + lax.axis_index("c")
    b = wid % 8
    if True:
        pltpu.sync_copy(x_hbm.at[b, 0], xb.at[pl.ds(PAD, L)])

        neg1 = jnp.full((NLANE,), -1.0, dtype=jnp.float32)
        zero = jnp.zeros((NLANE,), dtype=jnp.float32)

        def init_ab(t, _):
            ab[pl.ds(t * NLANE, NLANE)] = neg1
            bb[pl.ds(t * NLANE, NLANE)] = neg1
            kb[pl.ds(t * NLANE, NLANE)] = zero
            return 0

        lax.fori_loop(0, PL // NLANE, init_ab, 0)

        def init_vpad_lo(t, _):
            vb[pl.ds(t * NLANE, NLANE)] = neg1
            return 0

        lax.fori_loop(0, PAD // NLANE, init_vpad_lo, 0)

        def init_vpad_hi(t, _):
            vb[pl.ds(PAD + L + t * NLANE, NLANE)] = neg1
            return 0

        lax.fori_loop(0, (PL - PAD - L) // NLANE, init_vpad_hi, 0)

        xb[pl.ds(PAD - NLANE, NLANE)] = jnp.full((NLANE,), jnp.inf, jnp.float32)
        xb[pl.ds(PAD + L, NLANE)] = jnp.full((NLANE,), -jnp.inf, jnp.float32)

        def detect(t, _):
            base = PAD + t * NLANE
            xm = xb[pl.ds(base - 1, NLANE)]
            xc = xb[pl.ds(base, NLANE)]
            xp = xb[pl.ds(base + 1, NLANE)]
            dxr = (xp - xc) > 0
            dxl = (xc - xm) <= 0
            ndxr = (xp - xc) <= 0
            ndxl = (xc - xm) > 0
            valley = dxr & dxl & (xc <= 0)
            peak = ndxr & ndxl & (xc > 0)
            ex = valley | peak
            vb[pl.ds(base, NLANE)] = jnp.where(ex, jnp.abs(xc), -1.0)
            return 0

        lax.fori_loop(0, L // NLANE, detect, 0)

        niter = (L + 2 * D) // NLANE + 2

        def run64(src):
            bufs = [ab, bb]
            cur = src
            for p, s in enumerate((1, 2, 4, 8, 16, 32)):
                dst = bufs[p % 2]
                src_ref = cur

                def pass_body(t, _, src_ref=src_ref, dst=dst, s=s):
                    j = D + t * NLANE
                    a = src_ref[pl.ds(j, NLANE)]
                    c = src_ref[pl.ds(j + s, NLANE)]
                    dst[pl.ds(j, NLANE)] = jnp.maximum(a, c)
                    return 0

                lax.fori_loop(0, niter, pass_body, 0)
                cur = dst
            return cur

        def round_body():
            r64 = run64(vb)

            def keep(t, _):
                i0 = PAD + t * NLANE
                w = jnp.maximum(r64[pl.ds(i0 - D, NLANE)], vb[pl.ds(i0 + D, NLANE)])
                v = vb[pl.ds(i0, NLANE)]
                kn = (v >= 0) & (v >= w)
                kb[pl.ds(i0, NLANE)] = jnp.where(kn, 1.0, kb[pl.ds(i0, NLANE)])
                return 0

            lax.fori_loop(0, L // NLANE, keep, 0)

            r64k = run64(kb)

            def suppress(t, acc):
                i0 = PAD + t * NLANE
                s = jnp.maximum(r64k[pl.ds(i0 - D, NLANE)], kb[pl.ds(i0 + D, NLANE)])
                v = vb[pl.ds(i0, NLANE)]
                vn = jnp.where(s > 0, -1.0, v)
                vb[pl.ds(i0, NLANE)] = vn
                return jnp.maximum(acc, vn)

            acc = lax.fori_loop(0, L // NLANE, suppress, neg1)
            m = acc[0]
            for i in range(1, NLANE):
                m = jnp.maximum(m, acc[i])
            flag[0] = (m >= 0).astype(jnp.int32)

        flag[0] = jnp.int32(1)

        def outer(r, carry):
            @pl.when(flag[0] > 0)
            def _():
                round_body()

            return carry

        lax.fori_loop(0, 64, outer, 0)

        def emit(t, _):
            i0 = PAD + t * NLANE
            o = jnp.where(kb[pl.ds(i0, NLANE)] > 0, xb[pl.ds(i0, NLANE)], 0.0)
            ab[pl.ds(i0, NLANE)] = o
            return 0

        lax.fori_loop(0, L // NLANE, emit, 0)

        @pl.when(wid < 8)
        def _():
            pltpu.sync_copy(ab.at[pl.ds(PAD, L)], out_hbm.at[b, 0])


@jax.jit
def _nms(input_):
    mesh = plsc.VectorSubcoreMesh(core_axis_name="c", subcore_axis_name="s")
    f32 = jnp.float32
    return pl.kernel(
        _nms_body,
        mesh=mesh,
        out_type=jax.ShapeDtypeStruct((8, 1, L), f32),
        scratch_types=[
            pltpu.VMEM((PL,), f32),
            pltpu.VMEM((PL,), f32),
            pltpu.VMEM((PL,), f32),
            pltpu.VMEM((PL,), f32),
            pltpu.VMEM((PL,), f32),
            pltpu.SMEM((1,), jnp.int32),
        ],
    )(input_)


def kernel(input_):
    return _nms(input_)


# speedup vs baseline: 904.7159x; 2.0839x over previous
import functools

import jax
import jax.numpy as jnp
from jax import lax
from jax.experimental import pallas as pl
from jax.experimental.pallas import tpu as pltpu
from jax.experimental.pallas import tpu_sc as plsc

L = 16384
D = 32
PAD = 64
PL = L + 192
NLANE = 16
UNROLL = 8


def _nms_body(x_hbm, out_hbm, xb, vb, kb, ab, bb, flag):
    nc = 2
    wid = lax.axis_index("s") * nc + lax.axis_index("c")
    b = wid % 8
    pltpu.sync_copy(x_hbm.at[b, 0], xb.at[pl.ds(PAD, L)])

    neg1 = jnp.full((NLANE,), -1.0, dtype=jnp.float32)
    zero = jnp.zeros((NLANE,), dtype=jnp.float32)

    @plsc.parallel_loop(0, PL, step=NLANE, unroll=4)
    def _(j):
        ab[pl.ds(j, NLANE)] = neg1
        bb[pl.ds(j, NLANE)] = neg1
        kb[pl.ds(j, NLANE)] = zero

    @plsc.parallel_loop(0, PAD, step=NLANE, unroll=4)
    def _(j):
        vb[pl.ds(j, NLANE)] = neg1

    @plsc.parallel_loop(PAD + L, PL, step=NLANE, unroll=4)
    def _(j):
        vb[pl.ds(j, NLANE)] = neg1

    xb[pl.ds(PAD - NLANE, NLANE)] = jnp.full((NLANE,), jnp.inf, jnp.float32)
    xb[pl.ds(PAD + L, NLANE)] = jnp.full((NLANE,), -jnp.inf, jnp.float32)

    @plsc.parallel_loop(PAD, PAD + L, step=NLANE, unroll=UNROLL)
    def _(base):
        xm = xb[pl.ds(base - 1, NLANE)]
        xc = xb[pl.ds(base, NLANE)]
        xp = xb[pl.ds(base + 1, NLANE)]
        dxr = (xp - xc) > 0
        dxl = (xc - xm) <= 0
        ndxr = (xp - xc) <= 0
        ndxl = (xc - xm) > 0
        valley = dxr & dxl & (xc <= 0)
        peak = ndxr & ndxl & (xc > 0)
        ex = valley | peak
        vb[pl.ds(base, NLANE)] = jnp.where(ex, jnp.abs(xc), -1.0)

    hi = PAD + L + 2 * D + 2 * NLANE

    def run64(src):
        bufs = [ab, bb]
        cur = src
        for p, s in enumerate((1, 2, 4, 8, 16, 32)):
            dst = bufs[p % 2]
            src_ref = cur

            @plsc.parallel_loop(D, hi, step=NLANE, unroll=UNROLL)
            def _(j, src_ref=src_ref, dst=dst, s=s):
                a = src_ref[pl.ds(j, NLANE)]
                c = src_ref[pl.ds(j + s, NLANE)]
                dst[pl.ds(j, NLANE)] = jnp.maximum(a, c)

            cur = dst
        return cur

    def round_body():
        r64 = run64(vb)

        @plsc.parallel_loop(PAD, PAD + L, step=NLANE, unroll=UNROLL)
        def _(i0):
            w = jnp.maximum(r64[pl.ds(i0 - D, NLANE)], vb[pl.ds(i0 + D, NLANE)])
            v = vb[pl.ds(i0, NLANE)]
            kn = (v >= 0) & (v >= w)
            kb[pl.ds(i0, NLANE)] = jnp.where(kn, 1.0, kb[pl.ds(i0, NLANE)])

        r64k = run64(kb)

        @plsc.parallel_loop(PAD, PAD + L, step=NLANE, unroll=UNROLL, carry=neg1)
        def acc(i0, acc_in):
            s = jnp.maximum(r64k[pl.ds(i0 - D, NLANE)], kb[pl.ds(i0 + D, NLANE)])
            v = vb[pl.ds(i0, NLANE)]
            vn = jnp.where(s > 0, -1.0, v)
            vb[pl.ds(i0, NLANE)] = vn
            return jnp.maximum(acc_in, vn)

        m = acc[0]
        for i in range(1, NLANE):
            m = jnp.maximum(m, acc[i])
        flag[0] = (m >= 0).astype(jnp.int32)

    flag[0] = jnp.int32(1)

    def outer(r, carry):
        @pl.when(flag[0] > 0)
        def _():
            round_body()

        return carry

    lax.fori_loop(0, 64, outer, 0)

    @plsc.parallel_loop(PAD, PAD + L, step=NLANE, unroll=UNROLL)
    def _(i0):
        o = jnp.where(kb[pl.ds(i0, NLANE)] > 0, xb[pl.ds(i0, NLANE)], 0.0)
        ab[pl.ds(i0, NLANE)] = o

    @pl.when(wid < 8)
    def _():
        pltpu.sync_copy(ab.at[pl.ds(PAD, L)], out_hbm.at[b, 0])


@jax.jit
def _nms(input_):
    mesh = plsc.VectorSubcoreMesh(core_axis_name="c", subcore_axis_name="s")
    f32 = jnp.float32
    return pl.kernel(
        _nms_body,
        mesh=mesh,
        out_type=jax.ShapeDtypeStruct((8, 1, L), f32),
        scratch_types=[
            pltpu.VMEM((PL,), f32),
            pltpu.VMEM((PL,), f32),
            pltpu.VMEM((PL,), f32),
            pltpu.VMEM((PL,), f32),
            pltpu.VMEM((PL,), f32),
            pltpu.SMEM((1,), jnp.int32),
        ],
    )(input_)


def kernel(input_):
    return _nms(input_)
